# baseline (speedup 1.0000x reference)
import functools

import jax
import jax.numpy as jnp
from jax import lax
from jax.experimental import pallas as pl
from jax.experimental.pallas import tpu as pltpu
from jax.experimental.pallas import tpu_sc as plsc

N = 10000
E = 160000
D = 256
H = 128
ED = 16

NS = 16
K = 80
EPW = E // NS
CHUNKS = EPW // K
RPT = 624
REM = N - NS * RPT


def _edge_lin_body(ea_ref, w_ref, b_ref, lo_ref, hi_ref):
    e = jnp.dot(ea_ref[...], w_ref[...], preferred_element_type=jnp.float32)
    eb = (e + b_ref[...]).astype(jnp.bfloat16)
    bits = lax.bitcast_convert_type(eb, jnp.int16).astype(jnp.int32)
    lo_ref[...] = (bits[:, 0:64] & 0xFFFF) | (bits[:, 64:128] << 16)
    hi_ref[...] = (bits[:, 128:192] & 0xFFFF) | (bits[:, 192:256] << 16)


def _edge_lin(ea, w_t, b2d):
    be = 2000
    return pl.pallas_call(
        _edge_lin_body,
        grid=(E // be,),
        in_specs=[
            pl.BlockSpec((be, ED), lambda i: (i, 0)),
            pl.BlockSpec((ED, D), lambda i: (0, 0)),
            pl.BlockSpec((1, D), lambda i: (0, 0)),
        ],
        out_specs=[pl.BlockSpec((be, H // 2), lambda i: (i, 0))] * 2,
        out_shape=[jax.ShapeDtypeStruct((E, H // 2), jnp.int32)] * 2,
    )(ea, w_t, b2d)


def _widen_pairs(bits):
    even = lax.bitcast_convert_type(jnp.left_shift(bits, 16), jnp.float32)
    odd = lax.bitcast_convert_type(
        jnp.bitwise_and(bits, jnp.int32(-65536)), jnp.float32)
    return even, odd


def _sc_aggregate(src, dst, x_lo, x_hi, e_lo, e_hi, zeros):
    mesh = plsc.VectorSubcoreMesh(core_axis_name="c", subcore_axis_name="s")

    @functools.partial(
        pl.kernel,
        out_type=[jax.ShapeDtypeStruct((N, H), jnp.float32)] * 2,
        mesh=mesh,
        scratch_types=[
            pltpu.VMEM((2, K), jnp.int32),
            pltpu.VMEM((2, K), jnp.int32),
            pltpu.VMEM((2, K // 2, H), jnp.int32),
            pltpu.VMEM((2, K, H), jnp.float32),
            pltpu.VMEM((K, H), jnp.float32),
            pltpu.VMEM_SHARED((N, H), jnp.float32),
            pltpu.SemaphoreType.DMA,
            pltpu.SemaphoreType.DMA,
            pltpu.SemaphoreType.DMA,
            pltpu.SemaphoreType.DMA,
            pltpu.SemaphoreType.DMA,
            pltpu.SemaphoreType.DMA,
            pltpu.SemaphoreType.DMA,
            pltpu.SemaphoreType.DMA,
            pltpu.SemaphoreType.DMA,
            pltpu.SemaphoreType.DMA,
        ],
    )
    def sc_kernel(src_hbm, dst_hbm, xlo_hbm, xhi_hbm, elo_hbm, ehi_hbm,
                  z_hbm, outlo_hbm, outhi_hbm,
                  srcv, dstv, e_v, gx_v, msg_v, aggr_sh,
                  xsem0, xsem1, dsem0, dsem1, esem0, esem1,
                  gsem0, gsem1, ssem0, ssem1):
        c = lax.axis_index("c")
        s = lax.axis_index("s")
        xsem = (xsem0, xsem1)
        dsem = (dsem0, dsem1)
        esem = (esem0, esem1)
        gsem = (gsem0, gsem1)
        ssem = (ssem0, ssem1)

        rows = pl.ds(s * RPT, RPT)
        tail = pl.ds(NS * RPT, REM)
        ebase = s * EPW

        def run(x_tab, e_tab, out_tab):
            pltpu.sync_copy(z_hbm.at[pl.ds(0, RPT)], aggr_sh.at[rows])

            @pl.when(s == NS - 1)
            def _():
                pltpu.sync_copy(z_hbm.at[pl.ds(0, REM)], aggr_sh.at[tail])

            plsc.subcore_barrier()

            def load_src(ch, b):
                pltpu.async_copy(src_hbm.at[pl.ds(ebase + ch * K, K)],
                                 srcv.at[b], xsem[b])

            def load_dst(ch, b):
                pltpu.async_copy(dst_hbm.at[pl.ds(ebase + ch * K, K)],
                                 dstv.at[b], dsem[b])

            def fire_eg(ch, b):
                pltpu.make_async_copy(src_hbm.at[pl.ds(0, K)],
                                      srcv.at[b], xsem[b]).wait()
                pltpu.async_copy(
                    e_tab.at[pl.ds(s * (EPW // 2) + ch * (K // 2), K // 2)],
                    e_v.at[b], esem[b])
                pltpu.async_copy(x_tab.at[srcv.at[b]], gx_v.at[b], gsem[b])

            def process(ch, b, first=False):
                pltpu.make_async_copy(e_tab.at[pl.ds(0, K // 2)],
                                      e_v.at[b], esem[b]).wait()
                pltpu.make_async_copy(x_tab.at[srcv.at[0]],
                                      gx_v.at[b], gsem[b]).wait()
                if not first:
                    wait_scatter()

                def row(t, rc):
                    for u in range(2):
                        i = 2 * t + u
                        for j in range(H // 32):
                            ee, eo = _widen_pairs(
                                e_v[b, t, pl.ds(u * (H // 2) + j * 16, 16)])
                            lo = pl.ds(j * 16, 16)
                            hi = pl.ds(H // 2 + j * 16, 16)
                            msg_v[i, lo] = jnp.maximum(
                                ee + gx_v[b, i, lo], 0.0)
                            msg_v[i, hi] = jnp.maximum(
                                eo + gx_v[b, i, hi], 0.0)
                    return rc

                lax.fori_loop(0, K // 2, row, 0)
                pltpu.make_async_copy(dst_hbm.at[pl.ds(0, K)],
                                      dstv.at[b], dsem[b]).wait()
                pltpu.async_copy(msg_v, aggr_sh.at[dstv.at[b]],
                                 ssem0, add=True)

            def wait_scatter():
                pltpu.make_async_copy(msg_v, aggr_sh.at[dstv.at[0]],
                                      ssem0).wait()

            load_src(0, 0)
            load_src(1, 1)
            fire_eg(0, 0)
            load_dst(0, 0)
            fire_eg(1, 1)
            process(0, 0, first=True)
            load_src(2, 0)
            load_dst(1, 1)
            fire_eg(2, 0)
            process(1, 1)
            load_src(3, 1)

            def pair(kk, carry):
                i0 = 2 * kk + 2
                load_dst(i0, 0)
                fire_eg(i0 + 1, 1)
                process(i0, 0)
                load_src(i0 + 2, 0)

                i1 = i0 + 1
                load_dst(i1, 1)
                fire_eg(i1 + 1, 0)
                process(i1, 1)

                @pl.when(i1 + 2 < CHUNKS)
                def _():
                    load_src(i1 + 2, 1)

                return carry

            lax.fori_loop(0, (CHUNKS - 3) // 2, pair, 0)
            load_dst(CHUNKS - 1, 0)
            process(CHUNKS - 1, 0)
            wait_scatter()

            plsc.subcore_barrier()
            pltpu.sync_copy(aggr_sh.at[rows], out_tab.at[rows])

            @pl.when(s == NS - 1)
            def _():
                pltpu.sync_copy(aggr_sh.at[tail], out_tab.at[tail])

        @pl.when(c == 0)
        def _():
            run(xlo_hbm, elo_hbm, outlo_hbm)

        @pl.when(c == 1)
        def _():
            run(xhi_hbm, ehi_hbm, outhi_hbm)

    return sc_kernel(src, dst, x_lo, x_hi, e_lo, e_hi, zeros)


def _mlp_body(x_ref, alo_ref, ahi_ref, w1_ref, b1_ref, w2_ref, b2_ref, o_ref):
    a = jnp.concatenate([alo_ref[...], ahi_ref[...]], axis=1)
    h = x_ref[...] + a
    h = jnp.dot(h, w1_ref[...], preferred_element_type=jnp.float32) + b1_ref[...]
    h = jnp.maximum(h, 0.0)
    o_ref[...] = jnp.dot(h, w2_ref[...], preferred_element_type=jnp.float32) + b2_ref[...]


def _mlp(x, a_lo, a_hi, w1_t, b1_2d, w2_t, b2_2d):
    bn = 1000
    return pl.pallas_call(
        _mlp_body,
        grid=(N // bn,),
        in_specs=[
            pl.BlockSpec((bn, D), lambda i: (i, 0)),
            pl.BlockSpec((bn, H), lambda i: (i, 0)),
            pl.BlockSpec((bn, H), lambda i: (i, 0)),
            pl.BlockSpec((D, D), lambda i: (0, 0)),
            pl.BlockSpec((1, D), lambda i: (0, 0)),
            pl.BlockSpec((D, D), lambda i: (0, 0)),
            pl.BlockSpec((1, D), lambda i: (0, 0)),
        ],
        out_specs=pl.BlockSpec((bn, D), lambda i: (i, 0)),
        out_shape=jax.ShapeDtypeStruct((N, D), jnp.float32),
    )(x, a_lo, a_hi, w1_t, b1_2d, w2_t, b2_2d)


def kernel(x, edge_index, edge_attr, lin_W, lin_b, W1, b1, W2, b2):
    src = edge_index[0].astype(jnp.int32)
    dst = edge_index[1].astype(jnp.int32)
    e_lo, e_hi = _edge_lin(edge_attr, lin_W.T, lin_b.reshape(1, D))
    e_lo = e_lo.reshape(E // 2, H)
    e_hi = e_hi.reshape(E // 2, H)
    x_lo = x[:, :H]
    x_hi = x[:, H:]
    zeros = jnp.zeros((RPT, H), jnp.float32)
    a_lo, a_hi = _sc_aggregate(src, dst, x_lo, x_hi, e_lo, e_hi, zeros)
    return _mlp(x, a_lo, a_hi, W1.T, b1.reshape(1, D), W2.T, b2.reshape(1, D))


# speedup vs baseline: 1.6156x; 1.6156x over previous
import functools

import jax
import jax.numpy as jnp
from jax import lax
from jax.experimental import pallas as pl
from jax.experimental.pallas import tpu as pltpu
from jax.experimental.pallas import tpu_sc as plsc

N = 10000
E = 160000
D = 256
H = 128
ED = 16

NS = 16
K = 40
EPW = E // NS
CHUNKS = EPW // K
RPT = 624
REM = N - NS * RPT


def _edge_lin_body(ea_ref, w_ref, b_ref, lo_ref, hi_ref):
    e = jnp.dot(ea_ref[...], w_ref[...], preferred_element_type=jnp.float32)
    eb = (e + b_ref[...]).astype(jnp.bfloat16)
    bits = lax.bitcast_convert_type(eb, jnp.int16).astype(jnp.int32)
    lo_ref[...] = (bits[:, 0:64] & 0xFFFF) | (bits[:, 64:128] << 16)
    hi_ref[...] = (bits[:, 128:192] & 0xFFFF) | (bits[:, 192:256] << 16)


def _edge_lin(ea, w_t, b2d):
    be = 8000
    return pl.pallas_call(
        _edge_lin_body,
        grid=(E // be,),
        in_specs=[
            pl.BlockSpec((be, ED), lambda i: (i, 0)),
            pl.BlockSpec((ED, D), lambda i: (0, 0)),
            pl.BlockSpec((1, D), lambda i: (0, 0)),
        ],
        out_specs=[pl.BlockSpec((be, H // 2), lambda i: (i, 0))] * 2,
        out_shape=[jax.ShapeDtypeStruct((E, H // 2), jnp.int32)] * 2,
    )(ea, w_t, b2d)


def _widen_pairs(bits):
    even = lax.bitcast_convert_type(jnp.left_shift(bits, 16), jnp.float32)
    odd = lax.bitcast_convert_type(
        jnp.bitwise_and(bits, jnp.int32(-65536)), jnp.float32)
    return even, odd


def _sc_aggregate(src, dst, x_lo, x_hi, e_lo, e_hi, zeros):
    mesh = plsc.VectorSubcoreMesh(core_axis_name="c", subcore_axis_name="s")

    @functools.partial(
        pl.kernel,
        out_type=[jax.ShapeDtypeStruct((N, H), jnp.float32)] * 2,
        mesh=mesh,
        scratch_types=[
            pltpu.VMEM((2, K), jnp.int32),
            pltpu.VMEM((2, K), jnp.int32),
            pltpu.VMEM((2, K, H // 2), jnp.int32),
            pltpu.VMEM((2, K, H), jnp.float32),
            pltpu.VMEM((2, K, H), jnp.float32),
            pltpu.VMEM_SHARED((N, H), jnp.float32),
            pltpu.SemaphoreType.DMA,
            pltpu.SemaphoreType.DMA,
            pltpu.SemaphoreType.DMA,
            pltpu.SemaphoreType.DMA,
            pltpu.SemaphoreType.DMA,
            pltpu.SemaphoreType.DMA,
            pltpu.SemaphoreType.DMA,
            pltpu.SemaphoreType.DMA,
            pltpu.SemaphoreType.DMA,
            pltpu.SemaphoreType.DMA,
        ],
    )
    def sc_kernel(src_hbm, dst_hbm, xlo_hbm, xhi_hbm, elo_hbm, ehi_hbm,
                  z_hbm, outlo_hbm, outhi_hbm,
                  srcv, dstv, e_v, gx_v, msg_v, aggr_sh,
                  xsem0, xsem1, dsem0, dsem1, esem0, esem1,
                  gsem0, gsem1, ssem0, ssem1):
        c = lax.axis_index("c")
        s = lax.axis_index("s")
        xsem = (xsem0, xsem1)
        dsem = (dsem0, dsem1)
        esem = (esem0, esem1)
        gsem = (gsem0, gsem1)
        ssem = (ssem0, ssem1)

        rows = pl.ds(s * RPT, RPT)
        tail = pl.ds(NS * RPT, REM)
        ebase = s * EPW

        def run(x_tab, e_tab, out_tab):
            pltpu.sync_copy(z_hbm.at[pl.ds(0, RPT)], aggr_sh.at[rows])

            @pl.when(s == NS - 1)
            def _():
                pltpu.sync_copy(z_hbm.at[pl.ds(0, REM)], aggr_sh.at[tail])

            plsc.subcore_barrier()

            def load_src(ch, b):
                pltpu.async_copy(src_hbm.at[pl.ds(ebase + ch * K, K)],
                                 srcv.at[b], xsem[b])

            def load_dst(ch, b):
                pltpu.async_copy(dst_hbm.at[pl.ds(ebase + ch * K, K)],
                                 dstv.at[b], dsem[b])

            def fire_eg(ch, b):
                pltpu.make_async_copy(src_hbm.at[pl.ds(0, K)],
                                      srcv.at[b], xsem[b]).wait()
                pltpu.async_copy(e_tab.at[pl.ds(ebase + ch * K, K)],
                                 e_v.at[b], esem[b])
                pltpu.async_copy(x_tab.at[srcv.at[b]], gx_v.at[b], gsem[b])

            def process(ch, b):
                pltpu.make_async_copy(e_tab.at[pl.ds(0, K)],
                                      e_v.at[b], esem[b]).wait()
                pltpu.make_async_copy(x_tab.at[srcv.at[0]],
                                      gx_v.at[b], gsem[b]).wait()

                def row(i, rc):
                    for j in range(H // 32):
                        ee, eo = _widen_pairs(e_v[b, i, pl.ds(j * 16, 16)])
                        lo = pl.ds(j * 16, 16)
                        hi = pl.ds(H // 2 + j * 16, 16)
                        msg_v[b, i, lo] = jnp.maximum(
                            ee + gx_v[b, i, lo], 0.0)
                        msg_v[b, i, hi] = jnp.maximum(
                            eo + gx_v[b, i, hi], 0.0)
                    return rc

                lax.fori_loop(0, K, row, 0)
                pltpu.make_async_copy(dst_hbm.at[pl.ds(0, K)],
                                      dstv.at[b], dsem[b]).wait()
                pltpu.async_copy(msg_v.at[b], aggr_sh.at[dstv.at[b]],
                                 ssem[b], add=True)

            def wait_scatter(b):
                pltpu.make_async_copy(msg_v.at[b], aggr_sh.at[dstv.at[0]],
                                      ssem[b]).wait()

            load_src(0, 0)
            load_src(1, 1)
            fire_eg(0, 0)
            load_dst(0, 0)
            fire_eg(1, 1)
            process(0, 0)
            load_src(2, 0)
            load_dst(1, 1)
            fire_eg(2, 0)
            process(1, 1)
            load_src(3, 1)

            def pair(kk, carry):
                i0 = 2 * kk + 2
                wait_scatter(0)
                load_dst(i0, 0)
                fire_eg(i0 + 1, 1)
                process(i0, 0)

                @pl.when(i0 + 2 < CHUNKS)
                def _():
                    load_src(i0 + 2, 0)

                i1 = i0 + 1
                wait_scatter(1)
                load_dst(i1, 1)

                @pl.when(i1 + 1 < CHUNKS)
                def _():
                    fire_eg(i1 + 1, 0)

                process(i1, 1)

                @pl.when(i1 + 2 < CHUNKS)
                def _():
                    load_src(i1 + 2, 1)

                return carry

            lax.fori_loop(0, (CHUNKS - 2) // 2, pair, 0)
            wait_scatter(0)
            wait_scatter(1)

            plsc.subcore_barrier()
            pltpu.sync_copy(aggr_sh.at[rows], out_tab.at[rows])

            @pl.when(s == NS - 1)
            def _():
                pltpu.sync_copy(aggr_sh.at[tail], out_tab.at[tail])

        @pl.when(c == 0)
        def _():
            run(xlo_hbm, elo_hbm, outlo_hbm)

        @pl.when(c == 1)
        def _():
            run(xhi_hbm, ehi_hbm, outhi_hbm)

    return sc_kernel(src, dst, x_lo, x_hi, e_lo, e_hi, zeros)


def _mlp_body(x_ref, alo_ref, ahi_ref, w1_ref, b1_ref, w2_ref, b2_ref, o_ref):
    a = jnp.concatenate([alo_ref[...], ahi_ref[...]], axis=1)
    h = x_ref[...] + a
    h = jnp.dot(h, w1_ref[...], preferred_element_type=jnp.float32) + b1_ref[...]
    h = jnp.maximum(h, 0.0)
    o_ref[...] = jnp.dot(h, w2_ref[...], preferred_element_type=jnp.float32) + b2_ref[...]


def _mlp(x, a_lo, a_hi, w1_t, b1_2d, w2_t, b2_2d):
    bn = 2000
    return pl.pallas_call(
        _mlp_body,
        grid=(N // bn,),
        in_specs=[
            pl.BlockSpec((bn, D), lambda i: (i, 0)),
            pl.BlockSpec((bn, H), lambda i: (i, 0)),
            pl.BlockSpec((bn, H), lambda i: (i, 0)),
            pl.BlockSpec((D, D), lambda i: (0, 0)),
            pl.BlockSpec((1, D), lambda i: (0, 0)),
            pl.BlockSpec((D, D), lambda i: (0, 0)),
            pl.BlockSpec((1, D), lambda i: (0, 0)),
        ],
        out_specs=pl.BlockSpec((bn, D), lambda i: (i, 0)),
        out_shape=jax.ShapeDtypeStruct((N, D), jnp.float32),
    )(x, a_lo, a_hi, w1_t, b1_2d, w2_t, b2_2d)


def kernel(x, edge_index, edge_attr, lin_W, lin_b, W1, b1, W2, b2):
    src = edge_index[0].astype(jnp.int32)
    dst = edge_index[1].astype(jnp.int32)
    e_lo, e_hi = _edge_lin(edge_attr, lin_W.T, lin_b.reshape(1, D))
    x_lo = x[:, :H]
    x_hi = x[:, H:]
    zeros = jnp.zeros((RPT, H), jnp.float32)
    a_lo, a_hi = _sc_aggregate(src, dst, x_lo, x_hi, e_lo, e_hi, zeros)
    return _mlp(x, a_lo, a_hi, W1.T, b1.reshape(1, D), W2.T, b2.reshape(1, D))


# speedup vs baseline: 1.6334x; 1.0110x over previous
import functools

import jax
import jax.numpy as jnp
from jax import lax
from jax.experimental import pallas as pl
from jax.experimental.pallas import tpu as pltpu
from jax.experimental.pallas import tpu_sc as plsc

N = 10000
E = 160000
D = 256
H = 128
ED = 16

NS = 16
K = 40
EPW = E // NS
CHUNKS = EPW // K
RPT = 624
REM = N - NS * RPT


def _edge_lin_body(ea_ref, w_ref, b_ref, lo_ref, hi_ref):
    e = jnp.dot(ea_ref[...], w_ref[...], preferred_element_type=jnp.float32)
    eb = (e + b_ref[...]).astype(jnp.bfloat16)
    bits = lax.bitcast_convert_type(eb, jnp.int16).astype(jnp.int32)
    lo_ref[...] = (bits[:, 0:64] & 0xFFFF) | (bits[:, 64:128] << 16)
    hi_ref[...] = (bits[:, 128:192] & 0xFFFF) | (bits[:, 192:256] << 16)


def _edge_lin(ea, w_t, b2d):
    be = 16000
    return pl.pallas_call(
        _edge_lin_body,
        grid=(E // be,),
        in_specs=[
            pl.BlockSpec((be, ED), lambda i: (i, 0)),
            pl.BlockSpec((ED, D), lambda i: (0, 0)),
            pl.BlockSpec((1, D), lambda i: (0, 0)),
        ],
        out_specs=[pl.BlockSpec((be, H // 2), lambda i: (i, 0))] * 2,
        out_shape=[jax.ShapeDtypeStruct((E, H // 2), jnp.int32)] * 2,
    )(ea, w_t, b2d)


def _widen_pairs(bits):
    even = lax.bitcast_convert_type(jnp.left_shift(bits, 16), jnp.float32)
    odd = lax.bitcast_convert_type(
        jnp.bitwise_and(bits, jnp.int32(-65536)), jnp.float32)
    return even, odd


def _sc_aggregate(src, dst, x_lo, x_hi, e_lo, e_hi, zeros):
    mesh = plsc.VectorSubcoreMesh(core_axis_name="c", subcore_axis_name="s")

    @functools.partial(
        pl.kernel,
        out_type=[jax.ShapeDtypeStruct((N, H), jnp.float32)] * 2,
        mesh=mesh,
        scratch_types=[
            pltpu.VMEM((2, K), jnp.int32),
            pltpu.VMEM((2, K), jnp.int32),
            pltpu.VMEM((2, K, H // 2), jnp.int32),
            pltpu.VMEM((2, K, H), jnp.float32),
            pltpu.VMEM((2, K, H), jnp.float32),
            pltpu.VMEM_SHARED((N, H), jnp.float32),
            pltpu.SemaphoreType.DMA,
            pltpu.SemaphoreType.DMA,
            pltpu.SemaphoreType.DMA,
            pltpu.SemaphoreType.DMA,
            pltpu.SemaphoreType.DMA,
            pltpu.SemaphoreType.DMA,
            pltpu.SemaphoreType.DMA,
            pltpu.SemaphoreType.DMA,
            pltpu.SemaphoreType.DMA,
            pltpu.SemaphoreType.DMA,
        ],
    )
    def sc_kernel(src_hbm, dst_hbm, xlo_hbm, xhi_hbm, elo_hbm, ehi_hbm,
                  z_hbm, outlo_hbm, outhi_hbm,
                  srcv, dstv, e_v, gx_v, msg_v, aggr_sh,
                  xsem0, xsem1, dsem0, dsem1, esem0, esem1,
                  gsem0, gsem1, ssem0, ssem1):
        c = lax.axis_index("c")
        s = lax.axis_index("s")
        xsem = (xsem0, xsem1)
        dsem = (dsem0, dsem1)
        esem = (esem0, esem1)
        gsem = (gsem0, gsem1)
        ssem = (ssem0, ssem1)

        rows = pl.ds(s * RPT, RPT)
        tail = pl.ds(NS * RPT, REM)
        ebase = s * EPW

        def run(x_tab, e_tab, out_tab):
            pltpu.sync_copy(z_hbm.at[pl.ds(0, RPT)], aggr_sh.at[rows])

            @pl.when(s == NS - 1)
            def _():
                pltpu.sync_copy(z_hbm.at[pl.ds(0, REM)], aggr_sh.at[tail])

            plsc.subcore_barrier()

            def load_src(ch, b):
                pltpu.async_copy(src_hbm.at[pl.ds(ebase + ch * K, K)],
                                 srcv.at[b], xsem[b])

            def load_dst(ch, b):
                pltpu.async_copy(dst_hbm.at[pl.ds(ebase + ch * K, K)],
                                 dstv.at[b], dsem[b])

            def fire_eg(ch, b):
                pltpu.make_async_copy(src_hbm.at[pl.ds(0, K)],
                                      srcv.at[b], xsem[b]).wait()
                pltpu.async_copy(e_tab.at[pl.ds(ebase + ch * K, K)],
                                 e_v.at[b], esem[b])
                pltpu.async_copy(x_tab.at[srcv.at[b]], gx_v.at[b], gsem[b])

            def process(ch, b):
                pltpu.make_async_copy(e_tab.at[pl.ds(0, K)],
                                      e_v.at[b], esem[b]).wait()
                pltpu.make_async_copy(x_tab.at[srcv.at[0]],
                                      gx_v.at[b], gsem[b]).wait()

                def row(i, rc):
                    for j in range(H // 32):
                        ee, eo = _widen_pairs(e_v[b, i, pl.ds(j * 16, 16)])
                        lo = pl.ds(j * 16, 16)
                        hi = pl.ds(H // 2 + j * 16, 16)
                        msg_v[b, i, lo] = jnp.maximum(
                            ee + gx_v[b, i, lo], 0.0)
                        msg_v[b, i, hi] = jnp.maximum(
                            eo + gx_v[b, i, hi], 0.0)
                    return rc

                lax.fori_loop(0, K, row, 0)
                pltpu.make_async_copy(dst_hbm.at[pl.ds(0, K)],
                                      dstv.at[b], dsem[b]).wait()
                pltpu.async_copy(msg_v.at[b], aggr_sh.at[dstv.at[b]],
                                 ssem[b], add=True)

            def wait_scatter(b):
                pltpu.make_async_copy(msg_v.at[b], aggr_sh.at[dstv.at[0]],
                                      ssem[b]).wait()

            load_src(0, 0)
            load_src(1, 1)
            fire_eg(0, 0)
            load_dst(0, 0)
            fire_eg(1, 1)
            process(0, 0)
            load_src(2, 0)
            load_dst(1, 1)
            fire_eg(2, 0)
            process(1, 1)
            load_src(3, 1)

            def pair(kk, carry):
                i0 = 2 * kk + 2
                wait_scatter(0)
                load_dst(i0, 0)
                fire_eg(i0 + 1, 1)
                process(i0, 0)

                @pl.when(i0 + 2 < CHUNKS)
                def _():
                    load_src(i0 + 2, 0)

                i1 = i0 + 1
                wait_scatter(1)
                load_dst(i1, 1)

                @pl.when(i1 + 1 < CHUNKS)
                def _():
                    fire_eg(i1 + 1, 0)

                process(i1, 1)

                @pl.when(i1 + 2 < CHUNKS)
                def _():
                    load_src(i1 + 2, 1)

                return carry

            lax.fori_loop(0, (CHUNKS - 2) // 2, pair, 0)
            wait_scatter(0)
            wait_scatter(1)

            plsc.subcore_barrier()
            pltpu.sync_copy(aggr_sh.at[rows], out_tab.at[rows])

            @pl.when(s == NS - 1)
            def _():
                pltpu.sync_copy(aggr_sh.at[tail], out_tab.at[tail])

        @pl.when(c == 0)
        def _():
            run(xlo_hbm, elo_hbm, outlo_hbm)

        @pl.when(c == 1)
        def _():
            run(xhi_hbm, ehi_hbm, outhi_hbm)

    return sc_kernel(src, dst, x_lo, x_hi, e_lo, e_hi, zeros)


def _mlp_body(x_ref, alo_ref, ahi_ref, w1_ref, b1_ref, w2_ref, b2_ref, o_ref):
    a = jnp.concatenate([alo_ref[...], ahi_ref[...]], axis=1)
    h = x_ref[...] + a
    h = jnp.dot(h, w1_ref[...], preferred_element_type=jnp.float32) + b1_ref[...]
    h = jnp.maximum(h, 0.0)
    o_ref[...] = jnp.dot(h, w2_ref[...], preferred_element_type=jnp.float32) + b2_ref[...]


def _mlp(x, a_lo, a_hi, w1_t, b1_2d, w2_t, b2_2d):
    bn = 5000
    return pl.pallas_call(
        _mlp_body,
        grid=(N // bn,),
        in_specs=[
            pl.BlockSpec((bn, D), lambda i: (i, 0)),
            pl.BlockSpec((bn, H), lambda i: (i, 0)),
            pl.BlockSpec((bn, H), lambda i: (i, 0)),
            pl.BlockSpec((D, D), lambda i: (0, 0)),
            pl.BlockSpec((1, D), lambda i: (0, 0)),
            pl.BlockSpec((D, D), lambda i: (0, 0)),
            pl.BlockSpec((1, D), lambda i: (0, 0)),
        ],
        out_specs=pl.BlockSpec((bn, D), lambda i: (i, 0)),
        out_shape=jax.ShapeDtypeStruct((N, D), jnp.float32),
    )(x, a_lo, a_hi, w1_t, b1_2d, w2_t, b2_2d)


def kernel(x, edge_index, edge_attr, lin_W, lin_b, W1, b1, W2, b2):
    src = edge_index[0].astype(jnp.int32)
    dst = edge_index[1].astype(jnp.int32)
    e_lo, e_hi = _edge_lin(edge_attr, lin_W.T, lin_b.reshape(1, D))
    x_lo = x[:, :H]
    x_hi = x[:, H:]
    zeros = jnp.zeros((RPT, H), jnp.float32)
    a_lo, a_hi = _sc_aggregate(src, dst, x_lo, x_hi, e_lo, e_hi, zeros)
    return _mlp(x, a_lo, a_hi, W1.T, b1.reshape(1, D), W2.T, b2.reshape(1, D))
